# baseline (device time: 79011 ns/iter reference)
import jax
import jax.numpy as jnp
from jax import lax
from jax.experimental import pallas as pl
from jax.experimental.pallas import tpu as pltpu

N_DEV = 32
B, SQ, SKV, HQ_LOC, DH = 2, 256, 256, 4, 64
D_MODEL = 512
N_CHUNKS = 32
CHUNK_ROWS = (B * SQ) // N_CHUNKS


def kernel(x, Wq, K_ext, V_ext, Wo):
    me_out = lax.axis_index("i")
    k_loc = lax.dynamic_slice_in_dim(K_ext, me_out * HQ_LOC, HQ_LOC, axis=2)
    v_loc = lax.dynamic_slice_in_dim(V_ext, me_out * HQ_LOC, HQ_LOC, axis=2)

    def body(x_ref, wq_ref, k_ref, v_ref, wo_ref, out_ref,
             acc_ref, comm1_ref, comm2_ref, send1, recv1, send2, recv2):
        me = lax.axis_index("i")

        barrier = pltpu.get_barrier_semaphore()
        for o in range(N_DEV - 1):
            peer = lax.rem(me + 1 + o, N_DEV)
            pl.semaphore_signal(
                barrier, inc=1, device_id=(peer,),
                device_id_type=pl.DeviceIdType.MESH,
            )
        pl.semaphore_wait(barrier, N_DEV - 1)

        xb = x_ref[...].reshape(B * SQ, D_MODEL).astype(jnp.bfloat16)
        wq = wq_ref[...].astype(jnp.bfloat16)
        q = jnp.dot(xb, wq, preferred_element_type=jnp.float32)

        row_blk = lax.broadcasted_iota(jnp.int32, (SQ, SKV), 0) // 64
        col_blk = lax.broadcasted_iota(jnp.int32, (SQ, SKV), 1) // 64
        mask = col_blk <= row_blk

        ctx_rows = []
        for b in range(B):
            parts = []
            for h in range(HQ_LOC):
                qbh = q[b * SQ:(b + 1) * SQ, h * DH:(h + 1) * DH]
                qbh = qbh.astype(jnp.bfloat16)
                kbh = k_ref[b, :, h, :].astype(jnp.bfloat16)
                s = lax.dot_general(
                    qbh, kbh, (((1,), (1,)), ((), ())),
                    preferred_element_type=jnp.float32,
                ) * 0.125
                s = jnp.where(mask, s, -1e9)
                m = jnp.max(s, axis=-1, keepdims=True)
                w = jnp.exp(s - m)
                w = w / jnp.sum(w, axis=-1, keepdims=True)
                vbh = v_ref[b, :, h, :].astype(jnp.bfloat16)
                parts.append(jnp.dot(w.astype(jnp.bfloat16), vbh,
                                     preferred_element_type=jnp.float32))
            ctx_rows.append(jnp.concatenate(parts, axis=1))
        ctx = jnp.concatenate(ctx_rows, axis=0).astype(jnp.bfloat16)
        wo = wo_ref[...].astype(jnp.bfloat16)
        partial = jnp.dot(ctx, wo, preferred_element_type=jnp.float32)
        acc_ref[...] = partial.reshape(N_CHUNKS, CHUNK_ROWS, D_MODEL)

        rdmas1 = []
        for o in range(N_DEV - 1):
            j = lax.rem(me + 1 + o, N_DEV)
            r = pltpu.make_async_remote_copy(
                src_ref=acc_ref.at[j],
                dst_ref=comm1_ref.at[N_DEV - 2 - o],
                send_sem=send1.at[o],
                recv_sem=recv1.at[N_DEV - 2 - o],
                device_id=(j,),
                device_id_type=pl.DeviceIdType.MESH,
            )
            r.start()
            rdmas1.append(r)
        for r in rdmas1:
            r.wait_recv()
        reduced = acc_ref[me] + jnp.sum(comm1_ref[...], axis=0)
        acc_ref[me] = reduced
        out_ref[me] = reduced

        rdmas2 = []
        for o in range(N_DEV - 1):
            j = lax.rem(me + 1 + o, N_DEV)
            r = pltpu.make_async_remote_copy(
                src_ref=acc_ref.at[me],
                dst_ref=comm2_ref.at[N_DEV - 2 - o],
                send_sem=send2.at[o],
                recv_sem=recv2.at[N_DEV - 2 - o],
                device_id=(j,),
                device_id_type=pl.DeviceIdType.MESH,
            )
            r.start()
            rdmas2.append(r)
        for t in range(N_DEV - 1):
            rdmas2[N_DEV - 2 - t].wait_recv()
            s_idx = lax.rem(me + 1 + t, N_DEV)
            out_ref[s_idx] = comm2_ref[t]

        for r in rdmas1:
            r.wait_send()
        for r in rdmas2:
            r.wait_send()

    out = pl.pallas_call(
        body,
        out_shape=jax.ShapeDtypeStruct((N_CHUNKS, CHUNK_ROWS, D_MODEL),
                                       jnp.float32),
        in_specs=[pl.BlockSpec(memory_space=pltpu.VMEM)] * 5,
        out_specs=pl.BlockSpec(memory_space=pltpu.VMEM),
        scratch_shapes=[
            pltpu.VMEM((N_CHUNKS, CHUNK_ROWS, D_MODEL), jnp.float32),
            pltpu.VMEM((N_DEV - 1, CHUNK_ROWS, D_MODEL), jnp.float32),
            pltpu.VMEM((N_DEV - 1, CHUNK_ROWS, D_MODEL), jnp.float32),
            pltpu.SemaphoreType.DMA((N_DEV - 1,)),
            pltpu.SemaphoreType.DMA((N_DEV - 1,)),
            pltpu.SemaphoreType.DMA((N_DEV - 1,)),
            pltpu.SemaphoreType.DMA((N_DEV - 1,)),
        ],
        compiler_params=pltpu.CompilerParams(collective_id=0),
    )(x, Wq, k_loc, v_loc, Wo)
    return out.reshape(B, SQ, D_MODEL)


# device time: 73578 ns/iter; 1.0738x vs baseline; 1.0738x over previous
import jax
import jax.numpy as jnp
from jax import lax
from jax.experimental import pallas as pl
from jax.experimental.pallas import tpu as pltpu

N_DEV = 32
B, SQ, SKV, HQ_LOC, DH = 2, 256, 256, 4, 64
D_MODEL = 512
N_CHUNKS = 32
CHUNK_ROWS = (B * SQ) // N_CHUNKS


def kernel(x, Wq, K_ext, V_ext, Wo):
    def body(x_ref, wq_ref, k_hbm, v_hbm, wo_ref, out_ref,
             acc_ref, comm1_ref, comm2_ref, k_ref, v_ref, kv_sems,
             send1, recv1, send2, recv2):
        me = lax.axis_index("i")

        kcp = pltpu.make_async_copy(
            k_hbm.at[:, :, pl.ds(me * HQ_LOC, HQ_LOC), :], k_ref, kv_sems.at[0])
        vcp = pltpu.make_async_copy(
            v_hbm.at[:, :, pl.ds(me * HQ_LOC, HQ_LOC), :], v_ref, kv_sems.at[1])
        kcp.start()
        vcp.start()

        barrier = pltpu.get_barrier_semaphore()
        for o in range(N_DEV - 1):
            peer = lax.rem(me + 1 + o, N_DEV)
            pl.semaphore_signal(
                barrier, inc=1, device_id=(peer,),
                device_id_type=pl.DeviceIdType.MESH,
            )
        pl.semaphore_wait(barrier, N_DEV - 1)

        xb = x_ref[...].reshape(B * SQ, D_MODEL).astype(jnp.bfloat16)
        wq = wq_ref[...].astype(jnp.bfloat16)
        q = jnp.dot(xb, wq, preferred_element_type=jnp.float32)

        row_blk = lax.broadcasted_iota(jnp.int32, (SQ, SKV), 0) // 64
        col_blk = lax.broadcasted_iota(jnp.int32, (SQ, SKV), 1) // 64
        mask = col_blk <= row_blk

        kcp.wait()
        vcp.wait()
        ctx_rows = []
        for b in range(B):
            parts = []
            for h in range(HQ_LOC):
                qbh = q[b * SQ:(b + 1) * SQ, h * DH:(h + 1) * DH]
                qbh = qbh.astype(jnp.bfloat16)
                kbh = k_ref[b, :, h, :].astype(jnp.bfloat16)
                s = lax.dot_general(
                    qbh, kbh, (((1,), (1,)), ((), ())),
                    preferred_element_type=jnp.float32,
                ) * 0.125
                s = jnp.where(mask, s, -1e9)
                m = jnp.max(s, axis=-1, keepdims=True)
                w = jnp.exp(s - m)
                w = w / jnp.sum(w, axis=-1, keepdims=True)
                vbh = v_ref[b, :, h, :].astype(jnp.bfloat16)
                parts.append(jnp.dot(w.astype(jnp.bfloat16), vbh,
                                     preferred_element_type=jnp.float32))
            ctx_rows.append(jnp.concatenate(parts, axis=1))
        ctx = jnp.concatenate(ctx_rows, axis=0).astype(jnp.bfloat16)
        wo = wo_ref[...].astype(jnp.bfloat16)
        partial = jnp.dot(ctx, wo, preferred_element_type=jnp.float32)
        acc_ref[...] = partial.astype(jnp.bfloat16).reshape(
            N_CHUNKS, CHUNK_ROWS, D_MODEL)

        rdmas1 = []
        for o in range(N_DEV - 1):
            j = lax.rem(me + 1 + o, N_DEV)
            r = pltpu.make_async_remote_copy(
                src_ref=acc_ref.at[j],
                dst_ref=comm1_ref.at[N_DEV - 2 - o],
                send_sem=send1.at[o],
                recv_sem=recv1.at[N_DEV - 2 - o],
                device_id=(j,),
                device_id_type=pl.DeviceIdType.MESH,
            )
            r.start()
            rdmas1.append(r)
        for r in rdmas1:
            r.wait_recv()
        reduced = acc_ref[me].astype(jnp.float32) + jnp.sum(
            comm1_ref[...].astype(jnp.float32), axis=0)
        acc_ref[me] = reduced.astype(jnp.bfloat16)
        out_ref[me] = reduced

        rdmas2 = []
        for o in range(N_DEV - 1):
            j = lax.rem(me + 1 + o, N_DEV)
            r = pltpu.make_async_remote_copy(
                src_ref=acc_ref.at[me],
                dst_ref=comm2_ref.at[N_DEV - 2 - o],
                send_sem=send2.at[o],
                recv_sem=recv2.at[N_DEV - 2 - o],
                device_id=(j,),
                device_id_type=pl.DeviceIdType.MESH,
            )
            r.start()
            rdmas2.append(r)
        for t in range(N_DEV - 1):
            rdmas2[N_DEV - 2 - t].wait_recv()
            s_idx = lax.rem(me + 1 + t, N_DEV)
            out_ref[s_idx] = comm2_ref[t].astype(jnp.float32)

        for r in rdmas1:
            r.wait_send()
        for r in rdmas2:
            r.wait_send()

    out = pl.pallas_call(
        body,
        out_shape=jax.ShapeDtypeStruct((N_CHUNKS, CHUNK_ROWS, D_MODEL),
                                       jnp.float32),
        in_specs=[
            pl.BlockSpec(memory_space=pltpu.VMEM),
            pl.BlockSpec(memory_space=pltpu.VMEM),
            pl.BlockSpec(memory_space=pl.ANY),
            pl.BlockSpec(memory_space=pl.ANY),
            pl.BlockSpec(memory_space=pltpu.VMEM),
        ],
        out_specs=pl.BlockSpec(memory_space=pltpu.VMEM),
        scratch_shapes=[
            pltpu.VMEM((N_CHUNKS, CHUNK_ROWS, D_MODEL), jnp.bfloat16),
            pltpu.VMEM((N_DEV - 1, CHUNK_ROWS, D_MODEL), jnp.bfloat16),
            pltpu.VMEM((N_DEV - 1, CHUNK_ROWS, D_MODEL), jnp.bfloat16),
            pltpu.VMEM((B, SQ, HQ_LOC, DH), jnp.float32),
            pltpu.VMEM((B, SQ, HQ_LOC, DH), jnp.float32),
            pltpu.SemaphoreType.DMA((2,)),
            pltpu.SemaphoreType.DMA((N_DEV - 1,)),
            pltpu.SemaphoreType.DMA((N_DEV - 1,)),
            pltpu.SemaphoreType.DMA((N_DEV - 1,)),
            pltpu.SemaphoreType.DMA((N_DEV - 1,)),
        ],
        compiler_params=pltpu.CompilerParams(collective_id=0),
    )(x, Wq, K_ext, V_ext, Wo)
    return out.reshape(B, SQ, D_MODEL)


# device time: 57882 ns/iter; 1.3650x vs baseline; 1.2712x over previous
import jax
import jax.numpy as jnp
from jax import lax
from jax.experimental import pallas as pl
from jax.experimental.pallas import tpu as pltpu

N_DEV = 32
B, SQ, SKV, HQ_LOC, DH = 2, 256, 256, 4, 64
D_MODEL = 512
N_CHUNKS = 32
CHUNK_ROWS = (B * SQ) // N_CHUNKS


def kernel(x, Wq, K_ext, V_ext, Wo):
    me_out = lax.axis_index("i")
    k_loc = lax.dynamic_slice_in_dim(K_ext, me_out * HQ_LOC, HQ_LOC, axis=2)
    v_loc = lax.dynamic_slice_in_dim(V_ext, me_out * HQ_LOC, HQ_LOC, axis=2)
    k_loc = k_loc.reshape(B, SKV, HQ_LOC * DH).astype(jnp.bfloat16)
    v_loc = v_loc.reshape(B, SKV, HQ_LOC * DH).astype(jnp.bfloat16)

    def body(x_ref, wq_ref, k_ref, v_ref, wo_ref, out_ref,
             acc_ref, comm1_ref, comm2_ref, send1, recv1, send2, recv2):
        me = lax.axis_index("i")

        barrier = pltpu.get_barrier_semaphore()
        for o in range(N_DEV - 1):
            peer = lax.rem(me + 1 + o, N_DEV)
            pl.semaphore_signal(
                barrier, inc=1, device_id=(peer,),
                device_id_type=pl.DeviceIdType.MESH,
            )
        pl.semaphore_wait(barrier, N_DEV - 1)

        xb = x_ref[...].reshape(B * SQ, D_MODEL).astype(jnp.bfloat16)
        wq = wq_ref[...].astype(jnp.bfloat16)
        q = jnp.dot(xb, wq, preferred_element_type=jnp.float32)

        row_blk = lax.broadcasted_iota(jnp.int32, (SQ, SKV), 0) // 64
        col_blk = lax.broadcasted_iota(jnp.int32, (SQ, SKV), 1) // 64
        mask = col_blk <= row_blk

        ctx_rows = []
        for b in range(B):
            parts = []
            for h in range(HQ_LOC):
                qbh = q[b * SQ:(b + 1) * SQ, h * DH:(h + 1) * DH]
                qbh = qbh.astype(jnp.bfloat16)
                kbh = k_ref[b, :, h * DH:(h + 1) * DH]
                s = lax.dot_general(
                    qbh, kbh, (((1,), (1,)), ((), ())),
                    preferred_element_type=jnp.float32,
                ) * 0.125
                s = jnp.where(mask, s, -1e9)
                m = jnp.max(s, axis=-1, keepdims=True)
                w = jnp.exp(s - m)
                w = w / jnp.sum(w, axis=-1, keepdims=True)
                vbh = v_ref[b, :, h * DH:(h + 1) * DH]
                parts.append(jnp.dot(w.astype(jnp.bfloat16), vbh,
                                     preferred_element_type=jnp.float32))
            ctx_rows.append(jnp.concatenate(parts, axis=1))
        ctx = jnp.concatenate(ctx_rows, axis=0).astype(jnp.bfloat16)
        wo = wo_ref[...].astype(jnp.bfloat16)
        partial = jnp.dot(ctx, wo, preferred_element_type=jnp.float32)
        acc_ref[...] = partial.astype(jnp.bfloat16).reshape(
            N_CHUNKS, CHUNK_ROWS, D_MODEL)

        rdmas1 = []
        for o in range(N_DEV - 1):
            j = lax.rem(me + 1 + o, N_DEV)
            r = pltpu.make_async_remote_copy(
                src_ref=acc_ref.at[j],
                dst_ref=comm1_ref.at[N_DEV - 2 - o],
                send_sem=send1.at[o],
                recv_sem=recv1.at[N_DEV - 2 - o],
                device_id=(j,),
                device_id_type=pl.DeviceIdType.MESH,
            )
            r.start()
            rdmas1.append(r)
        for r in rdmas1:
            r.wait_recv()
        reduced = acc_ref[me].astype(jnp.float32) + jnp.sum(
            comm1_ref[...].astype(jnp.float32), axis=0)
        acc_ref[me] = reduced.astype(jnp.bfloat16)
        out_ref[me] = reduced

        rdmas2 = []
        for o in range(N_DEV - 1):
            j = lax.rem(me + 1 + o, N_DEV)
            r = pltpu.make_async_remote_copy(
                src_ref=acc_ref.at[me],
                dst_ref=comm2_ref.at[N_DEV - 2 - o],
                send_sem=send2.at[o],
                recv_sem=recv2.at[N_DEV - 2 - o],
                device_id=(j,),
                device_id_type=pl.DeviceIdType.MESH,
            )
            r.start()
            rdmas2.append(r)
        for t in range(N_DEV - 1):
            rdmas2[N_DEV - 2 - t].wait_recv()
            s_idx = lax.rem(me + 1 + t, N_DEV)
            out_ref[s_idx] = comm2_ref[t].astype(jnp.float32)

        for r in rdmas1:
            r.wait_send()
        for r in rdmas2:
            r.wait_send()

    out = pl.pallas_call(
        body,
        out_shape=jax.ShapeDtypeStruct((N_CHUNKS, CHUNK_ROWS, D_MODEL),
                                       jnp.float32),
        in_specs=[pl.BlockSpec(memory_space=pltpu.VMEM)] * 5,
        out_specs=pl.BlockSpec(memory_space=pltpu.VMEM),
        scratch_shapes=[
            pltpu.VMEM((N_CHUNKS, CHUNK_ROWS, D_MODEL), jnp.bfloat16),
            pltpu.VMEM((N_DEV - 1, CHUNK_ROWS, D_MODEL), jnp.bfloat16),
            pltpu.VMEM((N_DEV - 1, CHUNK_ROWS, D_MODEL), jnp.bfloat16),
            pltpu.SemaphoreType.DMA((N_DEV - 1,)),
            pltpu.SemaphoreType.DMA((N_DEV - 1,)),
            pltpu.SemaphoreType.DMA((N_DEV - 1,)),
            pltpu.SemaphoreType.DMA((N_DEV - 1,)),
        ],
        compiler_params=pltpu.CompilerParams(collective_id=0),
    )(x, Wq, k_loc, v_loc, Wo)
    return out.reshape(B, SQ, D_MODEL)


# device time: 57799 ns/iter; 1.3670x vs baseline; 1.0014x over previous
import jax
import jax.numpy as jnp
from jax import lax
from jax.experimental import pallas as pl
from jax.experimental.pallas import tpu as pltpu

N_DEV = 32
B, SQ, SKV, HQ_LOC, DH = 2, 256, 256, 4, 64
D_MODEL = 512
N_CHUNKS = 32
CHUNK_ROWS = (B * SQ) // N_CHUNKS


def kernel(x, Wq, K_ext, V_ext, Wo):
    me_out = lax.axis_index("i")
    k_loc = lax.dynamic_slice_in_dim(K_ext, me_out * HQ_LOC, HQ_LOC, axis=2)
    v_loc = lax.dynamic_slice_in_dim(V_ext, me_out * HQ_LOC, HQ_LOC, axis=2)
    k_loc = jnp.transpose(k_loc.astype(jnp.bfloat16), (0, 1, 3, 2)).reshape(
        B, SKV, HQ_LOC * DH)
    v_loc = jnp.transpose(v_loc.astype(jnp.bfloat16), (0, 1, 3, 2)).reshape(
        B, SKV, HQ_LOC * DH)

    def body(x_ref, wq_ref, k_ref, v_ref, wo_ref, out_ref,
             acc_ref, comm1_ref, comm2_ref, send1, recv1, send2, recv2):
        me = lax.axis_index("i")

        barrier = pltpu.get_barrier_semaphore()
        for o in range(N_DEV - 1):
            peer = lax.rem(me + 1 + o, N_DEV)
            pl.semaphore_signal(
                barrier, inc=1, device_id=(peer,),
                device_id_type=pl.DeviceIdType.MESH,
            )
        pl.semaphore_wait(barrier, N_DEV - 1)

        xb = x_ref[...].reshape(B * SQ, D_MODEL).astype(jnp.bfloat16)
        wq = wq_ref[...].astype(jnp.bfloat16)
        q = jnp.dot(xb, wq, preferred_element_type=jnp.float32)

        row_blk = lax.broadcasted_iota(jnp.int32, (SQ, SKV), 0) // 64
        col_blk = lax.broadcasted_iota(jnp.int32, (SQ, SKV), 1) // 64
        mask = col_blk <= row_blk

        pr = lax.broadcasted_iota(jnp.int32, (HQ_LOC * DH, HQ_LOC * DH), 0)
        pc = lax.broadcasted_iota(jnp.int32, (HQ_LOC * DH, HQ_LOC * DH), 1)
        perm = (pc == (pr % HQ_LOC) * DH + pr // HQ_LOC).astype(jnp.bfloat16)
        kc = [jnp.dot(k_ref[b], perm,
                      preferred_element_type=jnp.float32).astype(jnp.bfloat16)
              for b in range(B)]
        vc = [jnp.dot(v_ref[b], perm,
                      preferred_element_type=jnp.float32).astype(jnp.bfloat16)
              for b in range(B)]

        ctx_rows = []
        for b in range(B):
            parts = []
            for h in range(HQ_LOC):
                qbh = q[b * SQ:(b + 1) * SQ, h * DH:(h + 1) * DH]
                qbh = qbh.astype(jnp.bfloat16)
                kbh = kc[b][:, h * DH:(h + 1) * DH]
                s = lax.dot_general(
                    qbh, kbh, (((1,), (1,)), ((), ())),
                    preferred_element_type=jnp.float32,
                ) * 0.125
                s = jnp.where(mask, s, -1e9)
                m = jnp.max(s, axis=-1, keepdims=True)
                w = jnp.exp(s - m)
                w = w / jnp.sum(w, axis=-1, keepdims=True)
                vbh = vc[b][:, h * DH:(h + 1) * DH]
                parts.append(jnp.dot(w.astype(jnp.bfloat16), vbh,
                                     preferred_element_type=jnp.float32))
            ctx_rows.append(jnp.concatenate(parts, axis=1))
        ctx = jnp.concatenate(ctx_rows, axis=0).astype(jnp.bfloat16)
        wo = wo_ref[...].astype(jnp.bfloat16)
        partial = jnp.dot(ctx, wo, preferred_element_type=jnp.float32)
        acc_ref[...] = partial.astype(jnp.bfloat16).reshape(
            N_CHUNKS, CHUNK_ROWS, D_MODEL)

        rdmas1 = []
        for o in range(N_DEV - 1):
            j = lax.rem(me + 1 + o, N_DEV)
            r = pltpu.make_async_remote_copy(
                src_ref=acc_ref.at[j],
                dst_ref=comm1_ref.at[N_DEV - 2 - o],
                send_sem=send1.at[o],
                recv_sem=recv1.at[N_DEV - 2 - o],
                device_id=(j,),
                device_id_type=pl.DeviceIdType.MESH,
            )
            r.start()
            rdmas1.append(r)
        for r in rdmas1:
            r.wait_recv()
        reduced = acc_ref[me].astype(jnp.float32) + jnp.sum(
            comm1_ref[...].astype(jnp.float32), axis=0)
        acc_ref[me] = reduced.astype(jnp.bfloat16)
        out_ref[me] = reduced

        rdmas2 = []
        for o in range(N_DEV - 1):
            j = lax.rem(me + 1 + o, N_DEV)
            r = pltpu.make_async_remote_copy(
                src_ref=acc_ref.at[me],
                dst_ref=comm2_ref.at[N_DEV - 2 - o],
                send_sem=send2.at[o],
                recv_sem=recv2.at[N_DEV - 2 - o],
                device_id=(j,),
                device_id_type=pl.DeviceIdType.MESH,
            )
            r.start()
            rdmas2.append(r)
        for t in range(N_DEV - 1):
            rdmas2[N_DEV - 2 - t].wait_recv()
            s_idx = lax.rem(me + 1 + t, N_DEV)
            out_ref[s_idx] = comm2_ref[t].astype(jnp.float32)

        for r in rdmas1:
            r.wait_send()
        for r in rdmas2:
            r.wait_send()

    out = pl.pallas_call(
        body,
        out_shape=jax.ShapeDtypeStruct((N_CHUNKS, CHUNK_ROWS, D_MODEL),
                                       jnp.float32),
        in_specs=[pl.BlockSpec(memory_space=pltpu.VMEM)] * 5,
        out_specs=pl.BlockSpec(memory_space=pltpu.VMEM),
        scratch_shapes=[
            pltpu.VMEM((N_CHUNKS, CHUNK_ROWS, D_MODEL), jnp.bfloat16),
            pltpu.VMEM((N_DEV - 1, CHUNK_ROWS, D_MODEL), jnp.bfloat16),
            pltpu.VMEM((N_DEV - 1, CHUNK_ROWS, D_MODEL), jnp.bfloat16),
            pltpu.SemaphoreType.DMA((N_DEV - 1,)),
            pltpu.SemaphoreType.DMA((N_DEV - 1,)),
            pltpu.SemaphoreType.DMA((N_DEV - 1,)),
            pltpu.SemaphoreType.DMA((N_DEV - 1,)),
        ],
        compiler_params=pltpu.CompilerParams(collective_id=0),
    )(x, Wq, k_loc, v_loc, Wo)
    return out.reshape(B, SQ, D_MODEL)
